# Initial kernel scaffold; baseline (speedup 1.0000x reference)
import jax
import jax.numpy as jnp
from jax.experimental import pallas as pl


def kernel(x, edge_index, W1, b1, W2, b2):
    raise NotImplementedError("write your pallas kernel here")


# speedup vs baseline: 14.7201x; 14.7201x over previous
import functools

import jax
import jax.numpy as jnp
from jax import lax
from jax.experimental import pallas as pl
from jax.experimental.pallas import tpu as pltpu
from jax.experimental.pallas import tpu_sc as plsc

N = 10000
NPAD = 10240
E = 320000
CHUNK = 128
NCROWS = 2560
E_PAD = NCROWS * CHUNK
NC, NS = 2, 16
ROWS_16 = NCROWS // NS
ROWS_32 = NCROWS // (NC * NS)
BI = 16
RPT = NPAD // NS
IN_CH, HID, OUT_CH = 128, 256, 128
D = 128
BR = 256
GRID = NPAD // BR

f32 = jnp.float32
_mesh = plsc.VectorSubcoreMesh(core_axis_name="c", subcore_axis_name="s")


@functools.partial(
    pl.kernel,
    out_type=jax.ShapeDtypeStruct((NC, NPAD, D), f32),
    mesh=_mesh,
    scratch_types=[
        pltpu.VMEM((BI, CHUNK), jnp.int32),
        pltpu.VMEM((CHUNK, D), f32),
        pltpu.VMEM_SHARED((NPAD, D), f32),
    ],
)
def _deg_kernel(dst2d, ones128, zeros128, degp, dstv, ones_v, deg_sh):
    c = lax.axis_index("c")
    s = lax.axis_index("s")
    wid = c * NS + s
    pltpu.sync_copy(zeros128.at[pl.ds(s * RPT, RPT)],
                    deg_sh.at[pl.ds(s * RPT, RPT)])
    pltpu.sync_copy(ones128, ones_v)
    plsc.subcore_barrier()

    def block(b, carry):
        pltpu.sync_copy(dst2d.at[pl.ds(wid * ROWS_32 + b * BI, BI)], dstv)

        def step(i, carry2):
            pltpu.sync_copy(ones_v, deg_sh.at[dstv.at[i]], add=True)
            return carry2

        return lax.fori_loop(0, BI, step, carry)

    lax.fori_loop(0, ROWS_32 // BI, block, 0)
    plsc.subcore_barrier()
    pltpu.sync_copy(deg_sh.at[pl.ds(s * RPT, RPT)],
                    degp.at[c, pl.ds(s * RPT, RPT)])


def _edge_accumulate(tbl_hbm, srcs2d, dst2d, srcv, dstv, rows_v, agg_sh,
                     src_page, row_base, nrows):
    def block(b, carry):
        pltpu.sync_copy(srcs2d.at[src_page, pl.ds(row_base + b * BI, BI)],
                        srcv)
        pltpu.sync_copy(dst2d.at[pl.ds(row_base + b * BI, BI)], dstv)

        def step(i, carry2):
            pltpu.sync_copy(tbl_hbm.at[srcv.at[i]], rows_v)
            pltpu.sync_copy(rows_v, agg_sh.at[dstv.at[i]], add=True)
            return carry2

        return lax.fori_loop(0, BI, step, carry)

    lax.fori_loop(0, nrows // BI, block, 0)


@functools.partial(
    pl.kernel,
    out_type=jax.ShapeDtypeStruct((NC, NPAD, D), f32),
    mesh=_mesh,
    scratch_types=[
        pltpu.VMEM((BI, CHUNK), jnp.int32),
        pltpu.VMEM((BI, CHUNK), jnp.int32),
        pltpu.VMEM((CHUNK, D), f32),
        pltpu.VMEM_SHARED((NPAD, D), f32),
    ],
)
def _agg1_kernel(p_stack, srcs2d, dst2d, zeros128,
                 agg1, srcv, dstv, rows_v, agg_sh):
    c = lax.axis_index("c")
    s = lax.axis_index("s")
    pltpu.sync_copy(zeros128.at[pl.ds(s * RPT, RPT)],
                    agg_sh.at[pl.ds(s * RPT, RPT)])
    plsc.subcore_barrier()
    _edge_accumulate(p_stack, srcs2d, dst2d, srcv, dstv, rows_v, agg_sh,
                     c, s * ROWS_16, ROWS_16)
    plsc.subcore_barrier()
    pltpu.sync_copy(agg_sh.at[pl.ds(s * RPT, RPT)],
                    agg1.at[c, pl.ds(s * RPT, RPT)])


@functools.partial(
    pl.kernel,
    out_type=jax.ShapeDtypeStruct((NC, NPAD, D), f32),
    mesh=_mesh,
    scratch_types=[
        pltpu.VMEM((BI, CHUNK), jnp.int32),
        pltpu.VMEM((BI, CHUNK), jnp.int32),
        pltpu.VMEM((CHUNK, D), f32),
        pltpu.VMEM_SHARED((NPAD, D), f32),
    ],
)
def _agg2_kernel(p2, srcs2d, dst2d, zeros128,
                 agg2, srcv, dstv, rows_v, agg_sh):
    c = lax.axis_index("c")
    s = lax.axis_index("s")
    wid = c * NS + s
    pltpu.sync_copy(zeros128.at[pl.ds(s * RPT, RPT)],
                    agg_sh.at[pl.ds(s * RPT, RPT)])
    plsc.subcore_barrier()
    _edge_accumulate(p2, srcs2d, dst2d, srcv, dstv, rows_v, agg_sh,
                     0, wid * ROWS_32, ROWS_32)
    plsc.subcore_barrier()
    pltpu.sync_copy(agg_sh.at[pl.ds(s * RPT, RPT)],
                    agg2.at[c, pl.ds(s * RPT, RPT)])


def _dinv_block(degp_ref):
    deg = degp_ref[0, :, :1] + degp_ref[1, :, :1] + 1.0
    return lax.rsqrt(deg)


def _tc1_body(x_ref, w1_ref, degp_ref, p_ref):
    dinv = _dinv_block(degp_ref)
    h = jnp.dot(x_ref[...], w1_ref[...], preferred_element_type=f32)
    p_ref[...] = h * dinv


_tc1 = pl.pallas_call(
    _tc1_body,
    grid=(GRID, NC),
    in_specs=[
        pl.BlockSpec((BR, IN_CH), lambda i, j: (i, 0)),
        pl.BlockSpec((IN_CH, D), lambda i, j: (0, j)),
        pl.BlockSpec((NC, BR, D), lambda i, j: (0, i, 0)),
    ],
    out_specs=pl.BlockSpec((BR, D), lambda i, j: (j * GRID + i, 0)),
    out_shape=jax.ShapeDtypeStruct((NC * NPAD, D), f32),
)


def _tc2_body(agg1_ref, pa_ref, pb_ref, degp_ref, b1r_ref, w2_ref, p2_ref):
    dinv = _dinv_block(degp_ref)
    ya = jax.nn.relu(dinv * (agg1_ref[0] + pa_ref[...]) + b1r_ref[0:1, :])
    yb = jax.nn.relu(dinv * (agg1_ref[1] + pb_ref[...]) + b1r_ref[1:2, :])
    h2 = (jnp.dot(ya, w2_ref[:D, :], preferred_element_type=f32)
          + jnp.dot(yb, w2_ref[D:, :], preferred_element_type=f32))
    p2_ref[...] = h2 * dinv


_tc2 = pl.pallas_call(
    _tc2_body,
    grid=(GRID,),
    in_specs=[
        pl.BlockSpec((NC, BR, D), lambda i: (0, i, 0)),
        pl.BlockSpec((BR, D), lambda i: (i, 0)),
        pl.BlockSpec((BR, D), lambda i: (GRID + i, 0)),
        pl.BlockSpec((NC, BR, D), lambda i: (0, i, 0)),
        pl.BlockSpec((2, D), lambda i: (0, 0)),
        pl.BlockSpec((HID, OUT_CH), lambda i: (0, 0)),
    ],
    out_specs=pl.BlockSpec((BR, OUT_CH), lambda i: (i, 0)),
    out_shape=jax.ShapeDtypeStruct((NPAD, OUT_CH), f32),
)


def _tc3_body(agg2_ref, p2_ref, degp_ref, b2r_ref, o_ref):
    dinv = _dinv_block(degp_ref)
    o_ref[...] = dinv * (agg2_ref[0] + agg2_ref[1] + p2_ref[...]) \
        + b2r_ref[...]


_tc3 = pl.pallas_call(
    _tc3_body,
    grid=(GRID,),
    in_specs=[
        pl.BlockSpec((NC, BR, D), lambda i: (0, i, 0)),
        pl.BlockSpec((BR, D), lambda i: (i, 0)),
        pl.BlockSpec((NC, BR, D), lambda i: (0, i, 0)),
        pl.BlockSpec((1, OUT_CH), lambda i: (0, 0)),
    ],
    out_specs=pl.BlockSpec((BR, OUT_CH), lambda i: (i, 0)),
    out_shape=jax.ShapeDtypeStruct((NPAD, OUT_CH), f32),
)


def kernel(x, edge_index, W1, b1, W2, b2):
    src = edge_index[0].astype(jnp.int32)
    dst = edge_index[1].astype(jnp.int32)
    npe = E_PAD - E
    pad_src = (jnp.arange(npe, dtype=jnp.int32) * 131) % N
    pad_dst = (NPAD - 32) + (jnp.arange(npe, dtype=jnp.int32) % 32)
    src_f = jnp.concatenate([src, pad_src])
    srcs2d = jnp.stack([src_f, src_f + NPAD]).reshape(NC, NCROWS, CHUNK)
    dst2d = jnp.concatenate([dst, pad_dst]).reshape(NCROWS, CHUNK)

    x_pad = jnp.pad(x, ((0, NPAD - N), (0, 0)))
    zeros128 = jnp.zeros((NPAD, D), f32)
    ones128 = jnp.ones((CHUNK, D), f32)

    degp = _deg_kernel(dst2d, ones128, zeros128)
    p_stack = _tc1(x_pad, W1, degp)
    agg1 = _agg1_kernel(p_stack, srcs2d, dst2d, zeros128)
    p2 = _tc2(agg1, p_stack, p_stack, degp, b1.reshape(2, D), W2)
    agg2 = _agg2_kernel(p2, srcs2d, dst2d, zeros128)
    out = _tc3(agg2, p2, degp, b2.reshape(1, OUT_CH))
    return out[:N]


# speedup vs baseline: 17.8481x; 1.2125x over previous
import functools

import jax
import jax.numpy as jnp
from jax import lax
from jax.experimental import pallas as pl
from jax.experimental.pallas import tpu as pltpu
from jax.experimental.pallas import tpu_sc as plsc

N = 10000
NPAD = 10240
E = 320000
CHUNK = 128
NCROWS = 2560
E_PAD = NCROWS * CHUNK
NC, NS = 2, 16
ROWS_16 = NCROWS // NS
ROWS_32 = NCROWS // (NC * NS)
BI = 16
RPT = NPAD // NS
IN_CH, HID, OUT_CH = 128, 256, 128
D = 128
BR = 256
GRID = NPAD // BR

f32 = jnp.float32
_mesh = plsc.VectorSubcoreMesh(core_axis_name="c", subcore_axis_name="s")


@functools.partial(
    pl.kernel,
    out_type=jax.ShapeDtypeStruct((NC, NPAD, D), f32),
    mesh=_mesh,
    scratch_types=[
        pltpu.VMEM((BI, CHUNK), jnp.int32),
        pltpu.VMEM((CHUNK, D), f32),
        pltpu.VMEM_SHARED((NPAD, D), f32),
        pltpu.SemaphoreType.DMA,
    ],
)
def _deg_kernel(dst2d, ones128, zeros128, degp, dstv, ones_v, deg_sh, dsem):
    c = lax.axis_index("c")
    s = lax.axis_index("s")
    wid = c * NS + s
    pltpu.sync_copy(zeros128.at[pl.ds(s * RPT, RPT)],
                    deg_sh.at[pl.ds(s * RPT, RPT)])
    pltpu.sync_copy(ones128, ones_v)
    plsc.subcore_barrier()

    def block(b, carry):
        pltpu.sync_copy(dst2d.at[pl.ds(wid * ROWS_32 + b * BI, BI)], dstv)

        def fire(i, carry2):
            pltpu.async_copy(ones_v, deg_sh.at[dstv.at[i]], dsem, add=True)
            return carry2

        lax.fori_loop(0, BI, fire, 0)

        def drain(i, carry2):
            pltpu.make_async_copy(ones_v, deg_sh.at[dstv.at[i]], dsem).wait()
            return carry2

        return lax.fori_loop(0, BI, drain, carry)

    lax.fori_loop(0, ROWS_32 // BI, block, 0)
    plsc.subcore_barrier()
    pltpu.sync_copy(deg_sh.at[pl.ds(s * RPT, RPT)],
                    degp.at[c, pl.ds(s * RPT, RPT)])


def _edge_accumulate(tbl_hbm, srcs2d, dst2d, srcv, dstv, rows_v, agg_sh,
                     gs0, gs1, ss0, ss1, src_page, row_base, nrows):
    def gather(k, buf, sem):
        return pltpu.make_async_copy(tbl_hbm.at[srcv.at[k]],
                                     rows_v.at[buf], sem)

    def scat(k, buf, sem):
        return pltpu.make_async_copy(rows_v.at[buf],
                                     agg_sh.at[dstv.at[k]], sem)

    def block(b, carry):
        pltpu.sync_copy(srcs2d.at[src_page, pl.ds(row_base + b * BI, BI)],
                        srcv)
        pltpu.sync_copy(dst2d.at[pl.ds(row_base + b * BI, BI)], dstv)
        pltpu.async_copy(tbl_hbm.at[srcv.at[0]], rows_v.at[0], gs0)

        def pair(j, carry2):
            k0 = 2 * j
            k1 = k0 + 1
            gather(k0, 0, gs0).wait()
            pltpu.async_copy(rows_v.at[0], agg_sh.at[dstv.at[k0]], ss0,
                             add=True)

            @pl.when(j > 0)
            def _():
                scat(k0 - 1, 1, ss1).wait()

            pltpu.async_copy(tbl_hbm.at[srcv.at[k1]], rows_v.at[1], gs1)
            gather(k1, 1, gs1).wait()
            pltpu.async_copy(rows_v.at[1], agg_sh.at[dstv.at[k1]], ss1,
                             add=True)
            scat(k0, 0, ss0).wait()

            @pl.when(j < BI // 2 - 1)
            def _():
                pltpu.async_copy(tbl_hbm.at[srcv.at[k0 + 2]], rows_v.at[0],
                                 gs0)

            return carry2

        lax.fori_loop(0, BI // 2, pair, 0)
        scat(BI - 1, 1, ss1).wait()
        return carry

    lax.fori_loop(0, nrows // BI, block, 0)


@functools.partial(
    pl.kernel,
    out_type=jax.ShapeDtypeStruct((NC, NPAD, D), f32),
    mesh=_mesh,
    scratch_types=[
        pltpu.VMEM((BI, CHUNK), jnp.int32),
        pltpu.VMEM((BI, CHUNK), jnp.int32),
        pltpu.VMEM((2, CHUNK, D), f32),
        pltpu.VMEM_SHARED((NPAD, D), f32),
        pltpu.SemaphoreType.DMA,
        pltpu.SemaphoreType.DMA,
        pltpu.SemaphoreType.DMA,
        pltpu.SemaphoreType.DMA,
    ],
)
def _agg1_kernel(p_stack, srcs2d, dst2d, zeros128,
                 agg1, srcv, dstv, rows_v, agg_sh, gs0, gs1, ss0, ss1):
    c = lax.axis_index("c")
    s = lax.axis_index("s")
    pltpu.sync_copy(zeros128.at[pl.ds(s * RPT, RPT)],
                    agg_sh.at[pl.ds(s * RPT, RPT)])
    plsc.subcore_barrier()
    _edge_accumulate(p_stack, srcs2d, dst2d, srcv, dstv, rows_v, agg_sh,
                     gs0, gs1, ss0, ss1, c, s * ROWS_16, ROWS_16)
    plsc.subcore_barrier()
    pltpu.sync_copy(agg_sh.at[pl.ds(s * RPT, RPT)],
                    agg1.at[c, pl.ds(s * RPT, RPT)])


@functools.partial(
    pl.kernel,
    out_type=jax.ShapeDtypeStruct((NC, NPAD, D), f32),
    mesh=_mesh,
    scratch_types=[
        pltpu.VMEM((BI, CHUNK), jnp.int32),
        pltpu.VMEM((BI, CHUNK), jnp.int32),
        pltpu.VMEM((2, CHUNK, D), f32),
        pltpu.VMEM_SHARED((NPAD, D), f32),
        pltpu.SemaphoreType.DMA,
        pltpu.SemaphoreType.DMA,
        pltpu.SemaphoreType.DMA,
        pltpu.SemaphoreType.DMA,
    ],
)
def _agg2_kernel(p2, srcs2d, dst2d, zeros128,
                 agg2, srcv, dstv, rows_v, agg_sh, gs0, gs1, ss0, ss1):
    c = lax.axis_index("c")
    s = lax.axis_index("s")
    wid = c * NS + s
    pltpu.sync_copy(zeros128.at[pl.ds(s * RPT, RPT)],
                    agg_sh.at[pl.ds(s * RPT, RPT)])
    plsc.subcore_barrier()
    _edge_accumulate(p2, srcs2d, dst2d, srcv, dstv, rows_v, agg_sh,
                     gs0, gs1, ss0, ss1, 0, wid * ROWS_32, ROWS_32)
    plsc.subcore_barrier()
    pltpu.sync_copy(agg_sh.at[pl.ds(s * RPT, RPT)],
                    agg2.at[c, pl.ds(s * RPT, RPT)])


def _dinv_block(degp_ref):
    deg = degp_ref[0, :, :1] + degp_ref[1, :, :1] + 1.0
    return lax.rsqrt(deg)


def _tc1_body(x_ref, w1_ref, degp_ref, p_ref):
    dinv = _dinv_block(degp_ref)
    h = jnp.dot(x_ref[...], w1_ref[...], preferred_element_type=f32)
    p_ref[...] = h * dinv


_tc1 = pl.pallas_call(
    _tc1_body,
    grid=(GRID, NC),
    in_specs=[
        pl.BlockSpec((BR, IN_CH), lambda i, j: (i, 0)),
        pl.BlockSpec((IN_CH, D), lambda i, j: (0, j)),
        pl.BlockSpec((NC, BR, D), lambda i, j: (0, i, 0)),
    ],
    out_specs=pl.BlockSpec((BR, D), lambda i, j: (j * GRID + i, 0)),
    out_shape=jax.ShapeDtypeStruct((NC * NPAD, D), f32),
)


def _tc2_body(agg1_ref, pa_ref, pb_ref, degp_ref, b1r_ref, w2_ref, p2_ref):
    dinv = _dinv_block(degp_ref)
    ya = jax.nn.relu(dinv * (agg1_ref[0] + pa_ref[...]) + b1r_ref[0:1, :])
    yb = jax.nn.relu(dinv * (agg1_ref[1] + pb_ref[...]) + b1r_ref[1:2, :])
    h2 = (jnp.dot(ya, w2_ref[:D, :], preferred_element_type=f32)
          + jnp.dot(yb, w2_ref[D:, :], preferred_element_type=f32))
    p2_ref[...] = h2 * dinv


_tc2 = pl.pallas_call(
    _tc2_body,
    grid=(GRID,),
    in_specs=[
        pl.BlockSpec((NC, BR, D), lambda i: (0, i, 0)),
        pl.BlockSpec((BR, D), lambda i: (i, 0)),
        pl.BlockSpec((BR, D), lambda i: (GRID + i, 0)),
        pl.BlockSpec((NC, BR, D), lambda i: (0, i, 0)),
        pl.BlockSpec((2, D), lambda i: (0, 0)),
        pl.BlockSpec((HID, OUT_CH), lambda i: (0, 0)),
    ],
    out_specs=pl.BlockSpec((BR, OUT_CH), lambda i: (i, 0)),
    out_shape=jax.ShapeDtypeStruct((NPAD, OUT_CH), f32),
)


def _tc3_body(agg2_ref, p2_ref, degp_ref, b2r_ref, o_ref):
    dinv = _dinv_block(degp_ref)
    o_ref[...] = dinv * (agg2_ref[0] + agg2_ref[1] + p2_ref[...]) \
        + b2r_ref[...]


_tc3 = pl.pallas_call(
    _tc3_body,
    grid=(GRID,),
    in_specs=[
        pl.BlockSpec((NC, BR, D), lambda i: (0, i, 0)),
        pl.BlockSpec((BR, D), lambda i: (i, 0)),
        pl.BlockSpec((NC, BR, D), lambda i: (0, i, 0)),
        pl.BlockSpec((1, OUT_CH), lambda i: (0, 0)),
    ],
    out_specs=pl.BlockSpec((BR, OUT_CH), lambda i: (i, 0)),
    out_shape=jax.ShapeDtypeStruct((NPAD, OUT_CH), f32),
)


def kernel(x, edge_index, W1, b1, W2, b2):
    src = edge_index[0].astype(jnp.int32)
    dst = edge_index[1].astype(jnp.int32)
    npe = E_PAD - E
    pad_src = (jnp.arange(npe, dtype=jnp.int32) * 131) % N
    pad_dst = (NPAD - 32) + (jnp.arange(npe, dtype=jnp.int32) % 32)
    src_f = jnp.concatenate([src, pad_src])
    srcs2d = jnp.stack([src_f, src_f + NPAD]).reshape(NC, NCROWS, CHUNK)
    dst2d = jnp.concatenate([dst, pad_dst]).reshape(NCROWS, CHUNK)

    x_pad = jnp.pad(x, ((0, NPAD - N), (0, 0)))
    zeros128 = jnp.zeros((NPAD, D), f32)
    ones128 = jnp.ones((CHUNK, D), f32)

    degp = _deg_kernel(dst2d, ones128, zeros128)
    p_stack = _tc1(x_pad, W1, degp)
    agg1 = _agg1_kernel(p_stack, srcs2d, dst2d, zeros128)
    p2 = _tc2(agg1, p_stack, p_stack, degp, b1.reshape(2, D), W2)
    agg2 = _agg2_kernel(p2, srcs2d, dst2d, zeros128)
    out = _tc3(agg2, p2, degp, b2.reshape(1, OUT_CH))
    return out[:N]


# speedup vs baseline: 18.3815x; 1.0299x over previous
import functools

import jax
import jax.numpy as jnp
from jax import lax
from jax.experimental import pallas as pl
from jax.experimental.pallas import tpu as pltpu
from jax.experimental.pallas import tpu_sc as plsc

N = 10000
NPAD = 10240
E = 320000
CHUNK = 128
NCROWS = 2560
E_PAD = NCROWS * CHUNK
NC, NS = 2, 16
ROWS_16 = NCROWS // NS
ROWS_32 = NCROWS // (NC * NS)
BI = 40
BD = 16
RPT = NPAD // NS
IN_CH, HID, OUT_CH = 128, 256, 128
D = 128
BR = 256
GRID = NPAD // BR

f32 = jnp.float32
_mesh = plsc.VectorSubcoreMesh(core_axis_name="c", subcore_axis_name="s")


@functools.partial(
    pl.kernel,
    out_type=jax.ShapeDtypeStruct((NC, NPAD, D), f32),
    mesh=_mesh,
    scratch_types=[
        pltpu.VMEM((BD, CHUNK), jnp.int32),
        pltpu.VMEM((CHUNK, D), f32),
        pltpu.VMEM_SHARED((NPAD, D), f32),
        pltpu.SemaphoreType.DMA,
    ],
)
def _deg_kernel(dst2d, ones128, zeros128, degp, dstv, ones_v, deg_sh, dsem):
    c = lax.axis_index("c")
    s = lax.axis_index("s")
    wid = c * NS + s
    pltpu.sync_copy(zeros128.at[pl.ds(s * RPT, RPT)],
                    deg_sh.at[pl.ds(s * RPT, RPT)])
    pltpu.sync_copy(ones128, ones_v)
    plsc.subcore_barrier()

    def block(b, carry):
        pltpu.sync_copy(dst2d.at[pl.ds(wid * ROWS_32 + b * BD, BD)], dstv)

        def fire(i, carry2):
            pltpu.async_copy(ones_v, deg_sh.at[dstv.at[i]], dsem, add=True)
            return carry2

        lax.fori_loop(0, BD, fire, 0)

        def drain(i, carry2):
            pltpu.make_async_copy(ones_v, deg_sh.at[dstv.at[i]], dsem).wait()
            return carry2

        return lax.fori_loop(0, BD, drain, carry)

    lax.fori_loop(0, ROWS_32 // BD, block, 0)
    plsc.subcore_barrier()
    pltpu.sync_copy(deg_sh.at[pl.ds(s * RPT, RPT)],
                    degp.at[c, pl.ds(s * RPT, RPT)])


def _edge_accumulate(tbl_hbm, srcs2d, dst2d, srcv, dstv, rows_v, agg_sh,
                     gs0, gs1, ss0, ss1, src_page, row_base, nrows):
    def gather(k, buf, sem):
        return pltpu.make_async_copy(tbl_hbm.at[srcv.at[k]],
                                     rows_v.at[buf], sem)

    def scat(k, buf, sem):
        return pltpu.make_async_copy(rows_v.at[buf],
                                     agg_sh.at[dstv.at[k]], sem)

    def block(b, carry):
        pltpu.sync_copy(srcs2d.at[src_page, pl.ds(row_base + b * BI, BI)],
                        srcv)
        pltpu.sync_copy(dst2d.at[pl.ds(row_base + b * BI, BI)], dstv)
        pltpu.async_copy(tbl_hbm.at[srcv.at[0]], rows_v.at[0], gs0)

        def pair(j, carry2):
            k0 = 2 * j
            k1 = k0 + 1
            gather(k0, 0, gs0).wait()
            pltpu.async_copy(rows_v.at[0], agg_sh.at[dstv.at[k0]], ss0,
                             add=True)

            @pl.when(j > 0)
            def _():
                scat(k0 - 1, 1, ss1).wait()

            pltpu.async_copy(tbl_hbm.at[srcv.at[k1]], rows_v.at[1], gs1)
            gather(k1, 1, gs1).wait()
            pltpu.async_copy(rows_v.at[1], agg_sh.at[dstv.at[k1]], ss1,
                             add=True)
            scat(k0, 0, ss0).wait()

            @pl.when(j < BI // 2 - 1)
            def _():
                pltpu.async_copy(tbl_hbm.at[srcv.at[k0 + 2]], rows_v.at[0],
                                 gs0)

            return carry2

        lax.fori_loop(0, BI // 2, pair, 0)
        scat(BI - 1, 1, ss1).wait()
        return carry

    lax.fori_loop(0, nrows // BI, block, 0)


@functools.partial(
    pl.kernel,
    out_type=jax.ShapeDtypeStruct((NC, NPAD, D), f32),
    mesh=_mesh,
    scratch_types=[
        pltpu.VMEM((BI, CHUNK), jnp.int32),
        pltpu.VMEM((BI, CHUNK), jnp.int32),
        pltpu.VMEM((2, CHUNK, D), f32),
        pltpu.VMEM_SHARED((NPAD, D), f32),
        pltpu.SemaphoreType.DMA,
        pltpu.SemaphoreType.DMA,
        pltpu.SemaphoreType.DMA,
        pltpu.SemaphoreType.DMA,
    ],
)
def _agg1_kernel(p_stack, srcs2d, dst2d, zeros128,
                 agg1, srcv, dstv, rows_v, agg_sh, gs0, gs1, ss0, ss1):
    c = lax.axis_index("c")
    s = lax.axis_index("s")
    pltpu.sync_copy(zeros128.at[pl.ds(s * RPT, RPT)],
                    agg_sh.at[pl.ds(s * RPT, RPT)])
    plsc.subcore_barrier()
    _edge_accumulate(p_stack, srcs2d, dst2d, srcv, dstv, rows_v, agg_sh,
                     gs0, gs1, ss0, ss1, c, s * ROWS_16, ROWS_16)
    plsc.subcore_barrier()
    pltpu.sync_copy(agg_sh.at[pl.ds(s * RPT, RPT)],
                    agg1.at[c, pl.ds(s * RPT, RPT)])


@functools.partial(
    pl.kernel,
    out_type=jax.ShapeDtypeStruct((NC, NPAD, D), f32),
    mesh=_mesh,
    scratch_types=[
        pltpu.VMEM((BI, CHUNK), jnp.int32),
        pltpu.VMEM((BI, CHUNK), jnp.int32),
        pltpu.VMEM((2, CHUNK, D), f32),
        pltpu.VMEM_SHARED((NPAD, D), f32),
        pltpu.SemaphoreType.DMA,
        pltpu.SemaphoreType.DMA,
        pltpu.SemaphoreType.DMA,
        pltpu.SemaphoreType.DMA,
    ],
)
def _agg2_kernel(p2, srcs2d, dst2d, zeros128,
                 agg2, srcv, dstv, rows_v, agg_sh, gs0, gs1, ss0, ss1):
    c = lax.axis_index("c")
    s = lax.axis_index("s")
    wid = c * NS + s
    pltpu.sync_copy(zeros128.at[pl.ds(s * RPT, RPT)],
                    agg_sh.at[pl.ds(s * RPT, RPT)])
    plsc.subcore_barrier()
    _edge_accumulate(p2, srcs2d, dst2d, srcv, dstv, rows_v, agg_sh,
                     gs0, gs1, ss0, ss1, 0, wid * ROWS_32, ROWS_32)
    plsc.subcore_barrier()
    pltpu.sync_copy(agg_sh.at[pl.ds(s * RPT, RPT)],
                    agg2.at[c, pl.ds(s * RPT, RPT)])


def _dinv_block(degp_ref):
    deg = degp_ref[0, :, :1] + degp_ref[1, :, :1] + 1.0
    return lax.rsqrt(deg)


def _tc1a_body(x_ref, w1_ref, h_ref):
    h_ref[...] = jnp.dot(x_ref[...], w1_ref[...], preferred_element_type=f32)


_tc1a = pl.pallas_call(
    _tc1a_body,
    grid=(GRID, NC),
    in_specs=[
        pl.BlockSpec((BR, IN_CH), lambda i, j: (i, 0)),
        pl.BlockSpec((IN_CH, D), lambda i, j: (0, j)),
    ],
    out_specs=pl.BlockSpec((BR, D), lambda i, j: (j * GRID + i, 0)),
    out_shape=jax.ShapeDtypeStruct((NC * NPAD, D), f32),
)


def _tc1b_body(h_ref, degp_ref, p_ref):
    dinv = _dinv_block(degp_ref)
    p_ref[...] = h_ref[...] * dinv


_tc1b = pl.pallas_call(
    _tc1b_body,
    grid=(GRID, NC),
    in_specs=[
        pl.BlockSpec((BR, D), lambda i, j: (j * GRID + i, 0)),
        pl.BlockSpec((NC, BR, D), lambda i, j: (0, i, 0)),
    ],
    out_specs=pl.BlockSpec((BR, D), lambda i, j: (j * GRID + i, 0)),
    out_shape=jax.ShapeDtypeStruct((NC * NPAD, D), f32),
)


def _tc2_body(agg1_ref, pa_ref, pb_ref, degp_ref, b1r_ref, w2_ref, p2_ref):
    dinv = _dinv_block(degp_ref)
    ya = jax.nn.relu(dinv * (agg1_ref[0] + pa_ref[...]) + b1r_ref[0:1, :])
    yb = jax.nn.relu(dinv * (agg1_ref[1] + pb_ref[...]) + b1r_ref[1:2, :])
    h2 = (jnp.dot(ya, w2_ref[:D, :], preferred_element_type=f32)
          + jnp.dot(yb, w2_ref[D:, :], preferred_element_type=f32))
    p2_ref[...] = h2 * dinv


_tc2 = pl.pallas_call(
    _tc2_body,
    grid=(GRID,),
    in_specs=[
        pl.BlockSpec((NC, BR, D), lambda i: (0, i, 0)),
        pl.BlockSpec((BR, D), lambda i: (i, 0)),
        pl.BlockSpec((BR, D), lambda i: (GRID + i, 0)),
        pl.BlockSpec((NC, BR, D), lambda i: (0, i, 0)),
        pl.BlockSpec((2, D), lambda i: (0, 0)),
        pl.BlockSpec((HID, OUT_CH), lambda i: (0, 0)),
    ],
    out_specs=pl.BlockSpec((BR, OUT_CH), lambda i: (i, 0)),
    out_shape=jax.ShapeDtypeStruct((NPAD, OUT_CH), f32),
)


def _tc3_body(agg2_ref, p2_ref, degp_ref, b2r_ref, o_ref):
    dinv = _dinv_block(degp_ref)
    o_ref[...] = dinv * (agg2_ref[0] + agg2_ref[1] + p2_ref[...]) \
        + b2r_ref[...]


_tc3 = pl.pallas_call(
    _tc3_body,
    grid=(GRID,),
    in_specs=[
        pl.BlockSpec((NC, BR, D), lambda i: (0, i, 0)),
        pl.BlockSpec((BR, D), lambda i: (i, 0)),
        pl.BlockSpec((NC, BR, D), lambda i: (0, i, 0)),
        pl.BlockSpec((1, OUT_CH), lambda i: (0, 0)),
    ],
    out_specs=pl.BlockSpec((BR, OUT_CH), lambda i: (i, 0)),
    out_shape=jax.ShapeDtypeStruct((NPAD, OUT_CH), f32),
)


def kernel(x, edge_index, W1, b1, W2, b2):
    src = edge_index[0].astype(jnp.int32)
    dst = edge_index[1].astype(jnp.int32)
    npe = E_PAD - E
    pad_src = (jnp.arange(npe, dtype=jnp.int32) * 131) % N
    pad_dst = (NPAD - 32) + (jnp.arange(npe, dtype=jnp.int32) % 32)
    src_f = jnp.concatenate([src, pad_src])
    srcs2d = jnp.stack([src_f, src_f + NPAD]).reshape(NC, NCROWS, CHUNK)
    dst2d = jnp.concatenate([dst, pad_dst]).reshape(NCROWS, CHUNK)

    x_pad = jnp.pad(x, ((0, NPAD - N), (0, 0)))
    zeros128 = jnp.zeros((NPAD, D), f32)
    ones128 = jnp.ones((CHUNK, D), f32)

    degp = _deg_kernel(dst2d, ones128, zeros128)
    h_stack = _tc1a(x_pad, W1)
    p_stack = _tc1b(h_stack, degp)
    agg1 = _agg1_kernel(p_stack, srcs2d, dst2d, zeros128)
    p2 = _tc2(agg1, p_stack, p_stack, degp, b1.reshape(2, D), W2)
    agg2 = _agg2_kernel(p2, srcs2d, dst2d, zeros128)
    out = _tc3(agg2, p2, degp, b2.reshape(1, OUT_CH))
    return out[:N]
